# baseline (speedup 1.0000x reference)
import functools

import jax
import jax.numpy as jnp
from jax.experimental import pallas as pl
from jax.experimental.pallas import tpu as pltpu

BT = 512
K = 4


def _body(x_hbm, p_ref, g_ref, o_hbm, buf, obuf, isem, osem, *, scale, nsteps):
    def copy_in(i, slot):
        return pltpu.make_async_copy(
            x_hbm.at[pl.ds(i * BT, BT), :], buf.at[slot], isem.at[slot])

    def copy_out(i, slot):
        return pltpu.make_async_copy(
            obuf.at[slot], o_hbm.at[pl.ds(i * BT, BT), :], osem.at[slot])

    for s in range(min(K, nsteps)):
        copy_in(s, s).start()

    for i in range(nsteps):
        slot = i % K
        oslot = i % 2
        copy_in(i, slot).wait()
        acc = jax.lax.dot_general(
            buf[slot], p_ref[...],
            dimension_numbers=(((1,), (1,)), ((), ())),
            preferred_element_type=jnp.float32,
        )
        if i >= 2:
            copy_out(i - 2, oslot).wait()
        obuf[oslot] = jnp.maximum(acc * scale - g_ref[...], 0.0)
        copy_out(i, oslot).start()
        if i + K < nsteps:
            copy_in(i + K, slot).start()

    for i in range(max(nsteps - 2, 0), nsteps):
        copy_out(i, i % 2).wait()


def kernel(x, proto_k, gate):
    t, d = x.shape
    n = proto_k.shape[0]
    scale = 1.0 / (d ** 0.5)
    gate2d = gate.reshape(1, n)
    nsteps = t // BT
    return pl.pallas_call(
        functools.partial(_body, scale=scale, nsteps=nsteps),
        in_specs=[
            pl.BlockSpec(memory_space=pltpu.MemorySpace.HBM),
            pl.BlockSpec(memory_space=pltpu.MemorySpace.VMEM),
            pl.BlockSpec(memory_space=pltpu.MemorySpace.VMEM),
        ],
        out_specs=pl.BlockSpec(memory_space=pltpu.MemorySpace.HBM),
        out_shape=jax.ShapeDtypeStruct((t, n), jnp.float32),
        scratch_shapes=[
            pltpu.VMEM((K, BT, d), jnp.float32),
            pltpu.VMEM((2, BT, n), jnp.float32),
            pltpu.SemaphoreType.DMA((K,)),
            pltpu.SemaphoreType.DMA((2,)),
        ],
    )(x, proto_k, gate2d)


# speedup vs baseline: 1.0826x; 1.0826x over previous
import functools

import jax
import jax.numpy as jnp
from jax.experimental import pallas as pl
from jax.experimental.pallas import tpu as pltpu

BT = 512


def _body(x_ref, p_ref, g_ref, o_ref, *, scale):
    acc = jax.lax.dot_general(
        x_ref[...], p_ref[...],
        dimension_numbers=(((1,), (1,)), ((), ())),
        preferred_element_type=jnp.float32,
    )
    o_ref[...] = jnp.maximum(acc * scale - g_ref[...], 0.0)


def kernel(x, proto_k, gate):
    t, d = x.shape
    n = proto_k.shape[0]
    scale = 1.0 / (d ** 0.5)
    gate2d = gate.reshape(1, n)
    grid = (t // BT,)
    return pl.pallas_call(
        functools.partial(_body, scale=scale),
        grid=grid,
        in_specs=[
            pl.BlockSpec((BT, d), lambda i: (i, 0)),
            pl.BlockSpec((n, d), lambda i: (0, 0)),
            pl.BlockSpec((1, n), lambda i: (0, 0)),
        ],
        out_specs=pl.BlockSpec((BT, n), lambda i: (i, 0)),
        out_shape=jax.ShapeDtypeStruct((t, n), jnp.float32),
        compiler_params=pltpu.CompilerParams(
            dimension_semantics=("parallel",),
        ),
    )(x, proto_k, gate2d)
